# baseline (speedup 1.0000x reference)
import functools

import jax
import jax.numpy as jnp
from jax import lax
from jax.experimental import pallas as pl
from jax.experimental.pallas import tpu as pltpu
from jax.experimental.pallas import tpu_sc as plsc

_TOKENS = 16384
_IN_FEATURES = 2048
_NUM_EXPERTS = 64
_TOP_K = 8
_BLOCK_T = 2048

_NUM_WORKERS = 32
_TOK_PER_WORKER = _TOKENS // _NUM_WORKERS


def _router_block(x_ref, w_ref, scores_ref):
    x = x_ref[...]
    w = w_ref[...]
    logits = jax.lax.dot_general(
        x, w, (((1,), (1,)), ((), ())), preferred_element_type=jnp.float32
    )
    m = jnp.max(logits, axis=1, keepdims=True)
    e = jnp.exp(logits - m)
    s = jnp.sum(e, axis=1, keepdims=True)
    scores_ref[...] = e / s


def _tc_stage(x_video, W):
    return pl.pallas_call(
        _router_block,
        grid=(_TOKENS // _BLOCK_T,),
        in_specs=[
            pl.BlockSpec((_BLOCK_T, _IN_FEATURES), lambda t: (t, 0)),
            pl.BlockSpec((_NUM_EXPERTS, _IN_FEATURES), lambda t: (0, 0)),
        ],
        out_specs=pl.BlockSpec((_BLOCK_T, _NUM_EXPERTS), lambda t: (t, 0)),
        out_shape=jax.ShapeDtypeStruct((_TOKENS, _NUM_EXPERTS), jnp.float32),
        compiler_params=pltpu.CompilerParams(
            dimension_semantics=("arbitrary",),
        ),
    )(x_video, W)


def _rot8(x, rot8_idx):
    return lax.gather(
        x,
        rot8_idx[:, None],
        lax.GatherDimensionNumbers(
            offset_dims=(),
            collapsed_slice_dims=(0,),
            start_index_map=(0,),
        ),
        (1,),
        mode=lax.GatherScatterMode.PROMISE_IN_BOUNDS,
    )


def _sc_topk_body(scores_hbm, ts_hbm, ti_hbm, sc_v, ts_v, ti_v):
    wid = lax.axis_index("s") * 2 + lax.axis_index("c")
    base = wid * _TOK_PER_WORKER
    pltpu.sync_copy(
        scores_hbm.at[
            pl.ds(base * _NUM_EXPERTS, _TOK_PER_WORKER * _NUM_EXPERTS)
        ],
        sc_v,
    )

    lane = lax.iota(jnp.int32, 16)
    low8 = lane < 8
    rot8_idx = (lane + 8) & jnp.int32(15)
    group_iota = [lane + 16 * h for h in range(_NUM_EXPERTS // 16)]

    def merge(a, b):
        kbf = jnp.flip(b[0])
        vbf = jnp.flip(b[1])
        gt = a[0] >= kbf
        hk = jnp.where(gt, a[0], kbf)
        hv = jnp.where(gt, a[1], vbf)
        return plsc.sort_key_val(hk, hv, descending=True)

    def top8(tok):
        kb = tok * _NUM_EXPERTS
        parts = [
            plsc.sort_key_val(
                sc_v[pl.ds(kb + 16 * h, 16)], group_iota[h], descending=True
            )
            for h in range(_NUM_EXPERTS // 16)
        ]
        return merge(merge(parts[0], parts[1]), merge(parts[2], parts[3]))

    def quad(q, carry):
        for pp in range(4):
            tok_a = q * 8 + pp * 2
            ka, va = top8(tok_a)
            kb, vb = top8(tok_a + 1)
            ck = jnp.where(low8, ka, _rot8(kb, rot8_idx))
            cv = jnp.where(low8, va, _rot8(vb, rot8_idx))
            zero = jnp.float32(0.0)
            sa = jnp.sum(jnp.where(low8, ck, zero))
            sb = jnp.sum(jnp.where(low8, zero, ck))
            denom = jnp.where(low8, sa, sb) + jnp.float32(1e-6)
            ob = tok_a * _TOP_K
            ts_v[pl.ds(ob, 16)] = ck / denom
            ti_v[pl.ds(ob, 16)] = cv
        return carry

    lax.fori_loop(0, _TOK_PER_WORKER // 8, quad, 0)
    pltpu.sync_copy(
        ts_v, ts_hbm.at[pl.ds(base * _TOP_K, _TOK_PER_WORKER * _TOP_K)]
    )
    pltpu.sync_copy(
        ti_v, ti_hbm.at[pl.ds(base * _TOP_K, _TOK_PER_WORKER * _TOP_K)]
    )


def _sc_topk(scores_flat):
    mesh = plsc.VectorSubcoreMesh(core_axis_name="c", subcore_axis_name="s")
    fn = functools.partial(
        pl.kernel,
        mesh=mesh,
        out_type=(
            jax.ShapeDtypeStruct((_TOKENS * _TOP_K,), jnp.float32),
            jax.ShapeDtypeStruct((_TOKENS * _TOP_K,), jnp.int32),
        ),
        scratch_types=[
            pltpu.VMEM((_TOK_PER_WORKER * _NUM_EXPERTS,), jnp.float32),
            pltpu.VMEM((_TOK_PER_WORKER * _TOP_K,), jnp.float32),
            pltpu.VMEM((_TOK_PER_WORKER * _TOP_K,), jnp.int32),
        ],
        compiler_params=pltpu.CompilerParams(needs_layout_passes=False),
    )(_sc_topk_body)
    return fn(scores_flat)


def kernel(x_video, W):
    scores = _tc_stage(x_video, W)
    ts_flat, ti_flat = _sc_topk(scores.reshape(-1))
    topk_scores = ts_flat.reshape(_TOKENS, _TOP_K)
    topk_idx = ti_flat.reshape(_TOKENS, _TOP_K)
    return (scores, topk_scores, topk_idx)


# speedup vs baseline: 1.0547x; 1.0547x over previous
import functools

import jax
import jax.numpy as jnp
from jax import lax
from jax.experimental import pallas as pl
from jax.experimental.pallas import tpu as pltpu
from jax.experimental.pallas import tpu_sc as plsc

_TOKENS = 16384
_IN_FEATURES = 2048
_NUM_EXPERTS = 64
_TOP_K = 8
_BLOCK_T = 2048

_NUM_WORKERS = 32
_TOK_PER_WORKER = _TOKENS // _NUM_WORKERS


def _router_block(x_ref, w_ref, scores_ref):
    x = x_ref[...]
    w = w_ref[...]
    logits = jax.lax.dot_general(
        x, w, (((1,), (1,)), ((), ())), preferred_element_type=jnp.float32
    )
    m = jnp.max(logits, axis=1, keepdims=True)
    e = jnp.exp(logits - m)
    s = jnp.sum(e, axis=1, keepdims=True)
    scores_ref[...] = e / s


def _tc_stage(x_video, W):
    return pl.pallas_call(
        _router_block,
        grid=(_TOKENS // _BLOCK_T,),
        in_specs=[
            pl.BlockSpec((_BLOCK_T, _IN_FEATURES), lambda t: (t, 0)),
            pl.BlockSpec((_NUM_EXPERTS, _IN_FEATURES), lambda t: (0, 0)),
        ],
        out_specs=pl.BlockSpec((_BLOCK_T, _NUM_EXPERTS), lambda t: (t, 0)),
        out_shape=jax.ShapeDtypeStruct((_TOKENS, _NUM_EXPERTS), jnp.float32),
        compiler_params=pltpu.CompilerParams(
            dimension_semantics=("arbitrary",),
        ),
    )(x_video, W)


def _rot8(x, rot8_idx):
    return lax.gather(
        x,
        rot8_idx[:, None],
        lax.GatherDimensionNumbers(
            offset_dims=(),
            collapsed_slice_dims=(0,),
            start_index_map=(0,),
        ),
        (1,),
        mode=lax.GatherScatterMode.PROMISE_IN_BOUNDS,
    )


def _sc_topk_body(scores_hbm, ts_hbm, ti_hbm, sc_v, ts_v, ti_v):
    wid = lax.axis_index("s") * 2 + lax.axis_index("c")
    base = wid * _TOK_PER_WORKER
    pltpu.sync_copy(
        scores_hbm.at[
            pl.ds(base * _NUM_EXPERTS, _TOK_PER_WORKER * _NUM_EXPERTS)
        ],
        sc_v,
    )

    lane = lax.iota(jnp.int32, 16)
    low8 = lane < 8
    rot8_idx = (lane + 8) & jnp.int32(15)
    group_iota = [lane + 16 * h for h in range(_NUM_EXPERTS // 16)]

    def halfclean(a, b):
        gt = a[0] >= b[0]
        hk = jnp.where(gt, a[0], b[0])
        hv = jnp.where(gt, a[1], b[1])
        return hk, hv

    def top8(tok):
        kb = tok * _NUM_EXPERTS
        parts = [
            plsc.sort_key_val(
                sc_v[pl.ds(kb + 16 * h, 16)],
                group_iota[h],
                descending=(h % 2 == 1),
            )
            for h in range(_NUM_EXPERTS // 16)
        ]
        t01 = plsc.sort_key_val(*halfclean(parts[0], parts[1]))
        t23 = plsc.sort_key_val(
            *halfclean(parts[2], parts[3]), descending=True
        )
        return plsc.sort_key_val(*halfclean(t01, t23), descending=True)

    def quad(q, carry):
        for pp in range(2):
            tok_a = q * 4 + pp * 2
            ka, va = top8(tok_a)
            kb, vb = top8(tok_a + 1)
            ck = jnp.where(low8, ka, _rot8(kb, rot8_idx))
            cv = jnp.where(low8, va, _rot8(vb, rot8_idx))
            zero = jnp.float32(0.0)
            sa = jnp.sum(jnp.where(low8, ck, zero))
            sb = jnp.sum(jnp.where(low8, zero, ck))
            denom = jnp.where(low8, sa, sb) + jnp.float32(1e-6)
            ob = tok_a * _TOP_K
            ts_v[pl.ds(ob, 16)] = ck / denom
            ti_v[pl.ds(ob, 16)] = cv
        return carry

    lax.fori_loop(0, _TOK_PER_WORKER // 4, quad, 0)
    pltpu.sync_copy(
        ts_v, ts_hbm.at[pl.ds(base * _TOP_K, _TOK_PER_WORKER * _TOP_K)]
    )
    pltpu.sync_copy(
        ti_v, ti_hbm.at[pl.ds(base * _TOP_K, _TOK_PER_WORKER * _TOP_K)]
    )


def _sc_topk(scores_flat):
    mesh = plsc.VectorSubcoreMesh(core_axis_name="c", subcore_axis_name="s")
    fn = functools.partial(
        pl.kernel,
        mesh=mesh,
        out_type=(
            jax.ShapeDtypeStruct((_TOKENS * _TOP_K,), jnp.float32),
            jax.ShapeDtypeStruct((_TOKENS * _TOP_K,), jnp.int32),
        ),
        scratch_types=[
            pltpu.VMEM((_TOK_PER_WORKER * _NUM_EXPERTS,), jnp.float32),
            pltpu.VMEM((_TOK_PER_WORKER * _TOP_K,), jnp.float32),
            pltpu.VMEM((_TOK_PER_WORKER * _TOP_K,), jnp.int32),
        ],
        compiler_params=pltpu.CompilerParams(needs_layout_passes=False),
    )(_sc_topk_body)
    return fn(scores_flat)


def kernel(x_video, W):
    scores = _tc_stage(x_video, W)
    ts_flat, ti_flat = _sc_topk(scores.reshape(-1))
    topk_scores = ts_flat.reshape(_TOKENS, _TOP_K)
    topk_idx = ti_flat.reshape(_TOKENS, _TOP_K)
    return (scores, topk_scores, topk_idx)
